# baseline (speedup 1.0000x reference)
import functools

import jax
import jax.numpy as jnp
from jax.experimental import pallas as pl
from jax.experimental.pallas import tpu as pltpu

B = 16384
D = 512
K = 1000
ROWS = 512


def _body(f_ref, c_ref, d_ref, l_ref, p_ref):
    f = f_ref[...]
    c = c_ref[...]
    f2 = jnp.sum(f * f, axis=1, keepdims=True)
    c2 = jnp.sum(c * c, axis=1)[None, :]
    fc = jax.lax.dot_general(
        f, c, (((1,), (1,)), ((), ())),
        preferred_element_type=jnp.float32,
    )
    d2 = f2 + c2 - 2.0 * fc
    dists = jnp.sqrt(jnp.maximum(d2, 1e-12))
    d_ref[...] = dists
    l_ref[...] = -dists
    mn = jnp.min(dists, axis=1, keepdims=True)
    idx = jax.lax.broadcasted_iota(jnp.int32, dists.shape, 1)
    cand = jnp.where(dists == mn, idx, K)
    p_ref[...] = jnp.min(cand, axis=1, keepdims=True)


@jax.jit
def kernel(feat, centers):
    grid = (B // ROWS,)
    dists, logits, pred2d = pl.pallas_call(
        _body,
        grid=grid,
        in_specs=[
            pl.BlockSpec((ROWS, D), lambda i: (i, 0)),
            pl.BlockSpec((K, D), lambda i: (0, 0)),
        ],
        out_specs=[
            pl.BlockSpec((ROWS, K), lambda i: (i, 0)),
            pl.BlockSpec((ROWS, K), lambda i: (i, 0)),
            pl.BlockSpec((ROWS, 1), lambda i: (i, 0)),
        ],
        out_shape=[
            jax.ShapeDtypeStruct((B, K), jnp.float32),
            jax.ShapeDtypeStruct((B, K), jnp.float32),
            jax.ShapeDtypeStruct((B, 1), jnp.int32),
        ],
        compiler_params=pltpu.CompilerParams(
            dimension_semantics=("parallel",),
        ),
    )(feat, centers)
    return (dists, logits, pred2d.reshape(B))


# speedup vs baseline: 2.2158x; 2.2158x over previous
import jax
import jax.numpy as jnp
from jax.experimental import pallas as pl
from jax.experimental.pallas import tpu as pltpu

B = 16384
D = 512
K = 1000
COLS = 512


def _body(f_ref, c_ref, d_ref, l_ref, p_ref):
    f = f_ref[...]
    c = c_ref[...]
    ones = jnp.ones((1, D), jnp.float32)
    f2 = jax.lax.dot_general(
        ones, f * f, (((1,), (1,)), ((), ())),
        preferred_element_type=jnp.float32,
        precision=jax.lax.Precision.HIGHEST,
    )
    c2 = jnp.sum(c * c, axis=1, keepdims=True)
    cf = jax.lax.dot_general(
        c, f, (((1,), (1,)), ((), ())),
        preferred_element_type=jnp.float32,
    )
    d2 = f2 + c2 - 2.0 * cf
    dists = jnp.sqrt(jnp.maximum(d2, 1e-12))
    d_ref[...] = dists
    l_ref[...] = -dists
    mn = jnp.min(dists, axis=0, keepdims=True)
    idx = jax.lax.broadcasted_iota(jnp.int32, dists.shape, 0)
    cand = jnp.where(dists == mn, idx, K)
    p_ref[...] = jnp.min(cand, axis=0, keepdims=True)


@jax.jit
def kernel(feat, centers):
    grid = (B // COLS,)
    dists_t, logits_t, pred2d = pl.pallas_call(
        _body,
        grid=grid,
        in_specs=[
            pl.BlockSpec((COLS, D), lambda i: (i, 0)),
            pl.BlockSpec((K, D), lambda i: (0, 0)),
        ],
        out_specs=[
            pl.BlockSpec((K, COLS), lambda i: (0, i)),
            pl.BlockSpec((K, COLS), lambda i: (0, i)),
            pl.BlockSpec((1, COLS), lambda i: (0, i)),
        ],
        out_shape=[
            jax.ShapeDtypeStruct((K, B), jnp.float32),
            jax.ShapeDtypeStruct((K, B), jnp.float32),
            jax.ShapeDtypeStruct((1, B), jnp.int32),
        ],
        compiler_params=pltpu.CompilerParams(
            dimension_semantics=("arbitrary",),
        ),
    )(feat, centers)
    return (dists_t.T, logits_t.T, pred2d.reshape(B))


# speedup vs baseline: 2.7474x; 1.2399x over previous
import jax
import jax.numpy as jnp
from jax.experimental import pallas as pl
from jax.experimental.pallas import tpu as pltpu

B = 16384
D = 512
K = 1000
COLS = 512


def _body(f_ref, c_ref, d_ref, l_ref, p_ref):
    f = f_ref[...]
    c = c_ref[...]
    ones = jnp.ones((1, D), jnp.float32)
    f2 = jax.lax.dot_general(
        ones, f * f, (((1,), (1,)), ((), ())),
        preferred_element_type=jnp.float32,
    )
    c2 = jnp.sum(c * c, axis=1, keepdims=True)
    cf = jax.lax.dot_general(
        c, f, (((1,), (1,)), ((), ())),
        preferred_element_type=jnp.float32,
    )
    d2 = f2 + c2 - 2.0 * cf
    dists = jnp.sqrt(jnp.maximum(d2, 1e-12))
    d_ref[...] = dists
    l_ref[...] = -dists
    p_ref[...] = jnp.argmin(d2, axis=0)[None, :]


@jax.jit
def kernel(feat, centers):
    grid = (B // COLS,)
    dists_t, logits_t, pred2d = pl.pallas_call(
        _body,
        grid=grid,
        in_specs=[
            pl.BlockSpec((COLS, D), lambda i: (i, 0)),
            pl.BlockSpec((K, D), lambda i: (0, 0)),
        ],
        out_specs=[
            pl.BlockSpec((K, COLS), lambda i: (0, i)),
            pl.BlockSpec((K, COLS), lambda i: (0, i)),
            pl.BlockSpec((1, COLS), lambda i: (0, i)),
        ],
        out_shape=[
            jax.ShapeDtypeStruct((K, B), jnp.float32),
            jax.ShapeDtypeStruct((K, B), jnp.float32),
            jax.ShapeDtypeStruct((1, B), jnp.int32),
        ],
        compiler_params=pltpu.CompilerParams(
            dimension_semantics=("arbitrary",),
        ),
    )(feat, centers)
    return (dists_t.T, logits_t.T, pred2d.reshape(B))


# speedup vs baseline: 2.7926x; 1.0165x over previous
import jax
import jax.numpy as jnp
from jax.experimental import pallas as pl
from jax.experimental.pallas import tpu as pltpu

B = 16384
D = 512
K = 1000
COLS = 512


def _body(f_ref, c_ref, d_ref, l_ref, p_ref, c2_ref, cn_ref):
    @pl.when(pl.program_id(0) == 0)
    def _init():
        c = c_ref[...]
        c2_ref[...] = jnp.sum(c * c, axis=1, keepdims=True)
        cn_ref[...] = c * -2.0

    f = f_ref[...]
    ones = jnp.ones((1, D), jnp.float32)
    f2 = jax.lax.dot_general(
        ones, f * f, (((1,), (1,)), ((), ())),
        preferred_element_type=jnp.float32,
    )
    cf = jax.lax.dot_general(
        cn_ref[...], f, (((1,), (1,)), ((), ())),
        preferred_element_type=jnp.float32,
    )
    d2 = (f2 + c2_ref[...]) + cf
    dists = jnp.sqrt(jnp.maximum(d2, 1e-12))
    d_ref[...] = dists
    l_ref[...] = -dists
    p_ref[...] = jnp.argmin(d2, axis=0)[None, :]


@jax.jit
def kernel(feat, centers):
    grid = (B // COLS,)
    dists_t, logits_t, pred2d = pl.pallas_call(
        _body,
        grid=grid,
        in_specs=[
            pl.BlockSpec((COLS, D), lambda i: (i, 0)),
            pl.BlockSpec((K, D), lambda i: (0, 0)),
        ],
        out_specs=[
            pl.BlockSpec((K, COLS), lambda i: (0, i)),
            pl.BlockSpec((K, COLS), lambda i: (0, i)),
            pl.BlockSpec((1, COLS), lambda i: (0, i)),
        ],
        out_shape=[
            jax.ShapeDtypeStruct((K, B), jnp.float32),
            jax.ShapeDtypeStruct((K, B), jnp.float32),
            jax.ShapeDtypeStruct((1, B), jnp.int32),
        ],
        scratch_shapes=[
            pltpu.VMEM((K, 1), jnp.float32),
            pltpu.VMEM((K, D), jnp.float32),
        ],
        compiler_params=pltpu.CompilerParams(
            dimension_semantics=("arbitrary",),
        ),
    )(feat, centers)
    return (dists_t.T, logits_t.T, pred2d.reshape(B))
